# baseline (speedup 1.0000x reference)
import functools

import jax
import jax.numpy as jnp
from jax import lax
from jax.experimental import pallas as pl
from jax.experimental.pallas import tpu as pltpu
from jax.experimental.pallas import tpu_sc as plsc

N = 10000
D = 128
H = 256
E = 320000

NC = 2
NS = 16
L = 16

CH = 128
CPT = 160
GW = 16
NBUF = 2
SPROWS = 10016
EPT = CPT * CH
EPAD = NS * EPT
NCHT = EPAD // CH
NPAD = 11000
DEGP = 11008
ROWS_PT = 640
NB = 10
BR = N // NB

_mesh = plsc.VectorSubcoreMesh(
    core_axis_name="c", subcore_axis_name="s", num_cores=NC, num_subcores=NS)


@functools.partial(
    pl.kernel,
    out_type=jax.ShapeDtypeStruct((DEGP,), jnp.float32),
    mesh=_mesh,
    scratch_types=[
        pltpu.VMEM((CPT, CH), jnp.int32),
        pltpu.VMEM((CPT, CH), jnp.float32),
        pltpu.VMEM((DEGP // NS,), jnp.float32),
        pltpu.VMEM_SHARED((DEGP,), jnp.float32),
    ],
)
def _kdeg(dst_hbm, w_hbm, deg_hbm, dv, wv, zb, spdeg):
    c = lax.axis_index("c")
    s = lax.axis_index("s")

    @pl.when(c == 0)
    def _():
        sl = DEGP // NS
        for i in range(sl // L):
            zb[pl.ds(i * L, L)] = jnp.zeros((L,), jnp.float32)
        pltpu.sync_copy(zb, spdeg.at[pl.ds(s * sl, sl)])
        pltpu.sync_copy(dst_hbm.at[pl.ds(s * CPT, CPT)], dv)
        pltpu.sync_copy(w_hbm.at[pl.ds(s * CPT, CPT)], wv)
        plsc.subcore_barrier()

        def chunk(j, carry):
            pltpu.sync_copy(wv.at[j], spdeg.at[dv.at[j]], add=True)
            return carry

        lax.fori_loop(0, CPT, chunk, 0)
        plsc.subcore_barrier()
        pltpu.sync_copy(spdeg.at[pl.ds(s * sl, sl)], zb)
        pltpu.sync_copy(zb, deg_hbm.at[pl.ds(s * sl, sl)])


@functools.partial(
    pl.kernel,
    out_type=jax.ShapeDtypeStruct((NC * NPAD, 128), jnp.float32),
    mesh=_mesh,
    scratch_types=[
        pltpu.VMEM((GW, CH), jnp.int32),
        pltpu.VMEM((GW, CH), jnp.int32),
        pltpu.VMEM((GW, CH), jnp.float32),
        pltpu.VMEM((NBUF * CH, 128), jnp.float32),
        pltpu.VMEM_SHARED((SPROWS, 128), jnp.float32),
        pltpu.SemaphoreType.DMA,
        pltpu.SemaphoreType.DMA,
        pltpu.SemaphoreType.DMA,
        pltpu.SemaphoreType.DMA,
    ],
)
def _ksc(hw_hbm, gidx_hbm, dst_hbm, w_hbm, acc_hbm, gv, dv, wv, rbuf,
         spacc, s0, s1, s2, s3):
    sems = (s0, s1, s2, s3)
    c = lax.axis_index("c")
    s = lax.axis_index("s")
    rb = lambda b: rbuf.at[pl.ds(b * CH, CH)]
    rbase = jnp.minimum(s * ROWS_PT, N - ROWS_PT)
    for p in range(ROWS_PT // CH):
        pltpu.sync_copy(hw_hbm.at[pl.ds(c * N + rbase + p * CH, CH)], rb(p % NBUF))
        pltpu.sync_copy(rb(p % NBUF), spacc.at[pl.ds(rbase + p * CH, CH)])
    plsc.subcore_barrier()

    def issue(p_idx, b):
        pltpu.async_copy(hw_hbm.at[gv.at[p_idx, pl.ds(0, 64)]],
                         rbuf.at[pl.ds(b * CH, 64)], sems[2 * b])
        pltpu.async_copy(hw_hbm.at[gv.at[p_idx, pl.ds(64, 64)]],
                         rbuf.at[pl.ds(b * CH + 64, 64)], sems[2 * b + 1])

    def window(w, carry0):
        cbase = s * CPT + w * GW
        pltpu.sync_copy(gidx_hbm.at[pl.ds(c * NCHT + cbase, GW)], gv)
        pltpu.sync_copy(dst_hbm.at[pl.ds(cbase, GW)], dv)
        pltpu.sync_copy(w_hbm.at[pl.ds(cbase, GW)], wv)
        for b in range(NBUF):
            issue(b, b)

        for p in range(GW):
            b = p % NBUF

            def group(q, carry2, p=p, b=b):
                wrow = wv[p, pl.ds(q * L, L)]
                base = q * L
                for t in range(L):
                    wb = jnp.full((L,), wrow[t], jnp.float32)
                    e = base + t
                    for v in range(128 // L):
                        rbuf[b * CH + e, pl.ds(v * L, L)] = (
                            rbuf[b * CH + e, pl.ds(v * L, L)] * wb)
                return carry2

            pltpu.make_async_copy(hw_hbm.at[pl.ds(0, 64)],
                                  rbuf.at[pl.ds(b * CH, 64)],
                                  sems[2 * b]).wait()
            lax.fori_loop(0, CH // L // 2, group, 0)
            pltpu.make_async_copy(hw_hbm.at[pl.ds(0, 64)],
                                  rbuf.at[pl.ds(b * CH + 64, 64)],
                                  sems[2 * b + 1]).wait()
            lax.fori_loop(CH // L // 2, CH // L, group, 0)
            pltpu.sync_copy(rb(b), spacc.at[dv.at[p]], add=True)
            if p + NBUF < GW:
                issue(p + NBUF, b)

        return carry0

    lax.fori_loop(0, CPT // GW, window, 0)
    plsc.subcore_barrier()
    for p in range(ROWS_PT // CH):
        pltpu.sync_copy(spacc.at[pl.ds(rbase + p * CH, CH)], rb(p % NBUF))
        pltpu.sync_copy(rb(p % NBUF),
                        acc_hbm.at[pl.ds(c * NPAD + rbase + p * CH, CH)])


def _k0_body(x_ref, pos_ref, wpos_ref, bpos_ref, win_ref, bin_ref, o_ref):
    pe = jnp.dot(pos_ref[...], wpos_ref[...],
                 preferred_element_type=jnp.float32) + bpos_ref[...]
    h0 = (jnp.dot(x_ref[...], win_ref[0:D],
                  preferred_element_type=jnp.float32)
          + jnp.dot(pe, win_ref[D:D + H],
                    preferred_element_type=jnp.float32)
          + bin_ref[...])
    o_ref[...] = h0


def _k0(x, posp, wposp, bpos, win, bin_):
    return pl.pallas_call(
        _k0_body,
        grid=(NB,),
        in_specs=[
            pl.BlockSpec((BR, D), lambda i: (i, 0)),
            pl.BlockSpec((BR, 8), lambda i: (i, 0)),
            pl.BlockSpec((8, H), lambda i: (0, 0)),
            pl.BlockSpec((1, H), lambda i: (0, 0)),
            pl.BlockSpec((D + H, H), lambda i: (0, 0)),
            pl.BlockSpec((1, H), lambda i: (0, 0)),
        ],
        out_specs=pl.BlockSpec((BR, H), lambda i: (i, 0)),
        out_shape=jax.ShapeDtypeStruct((N, H), jnp.float32),
    )(x, posp, wposp, bpos, win, bin_)


def _mm1_body(h_ref, w_ref, deg_ref, o_ref):
    dis = lax.rsqrt(deg_ref[...] + 1.0)
    o_ref[...] = jnp.dot(h_ref[...], w_ref[...],
                         preferred_element_type=jnp.float32) * dis


def _mm1(h, W, deg):
    return pl.pallas_call(
        _mm1_body,
        grid=(NB, 2),
        in_specs=[
            pl.BlockSpec((BR, H), lambda i, j: (i, 0)),
            pl.BlockSpec((H, 128), lambda i, j: (0, j)),
            pl.BlockSpec((BR, 1), lambda i, j: (i, 0)),
        ],
        out_specs=pl.BlockSpec((BR, 128), lambda i, j: (j * NB + i, 0)),
        out_shape=jax.ShapeDtypeStruct((2 * N, 128), jnp.float32),
    )(h, W, deg)


def _k2a_body(acc_ref, deg_ref, b_ref, z_ref, p_ref):
    dis = lax.rsqrt(deg_ref[...] + 1.0)
    z = jnp.concatenate([acc_ref[0], acc_ref[1]], axis=1) * dis + b_ref[...]
    z_ref[...] = z
    s1 = jnp.sum(z, axis=0)
    s2 = jnp.sum(z * z, axis=0)
    p_ref[...] = jnp.stack([s1, s2])[None]


def _k2a(accv, deg, b):
    return pl.pallas_call(
        _k2a_body,
        grid=(NB,),
        in_specs=[
            pl.BlockSpec((2, BR, 128), lambda i: (0, i, 0)),
            pl.BlockSpec((BR, 1), lambda i: (i, 0)),
            pl.BlockSpec((1, H), lambda i: (0, 0)),
        ],
        out_specs=[
            pl.BlockSpec((BR, H), lambda i: (i, 0)),
            pl.BlockSpec((1, 2, H), lambda i: (i, 0, 0)),
        ],
        out_shape=[
            jax.ShapeDtypeStruct((N, H), jnp.float32),
            jax.ShapeDtypeStruct((NB, 2, H), jnp.float32),
        ],
    )(accv, deg, b)


def _bn_from_partials(p, g_ref, be_ref):
    mu = jnp.sum(p[:, 0, :], axis=0) * (1.0 / N)
    ms = jnp.sum(p[:, 1, :], axis=0) * (1.0 / N)
    var = ms - mu * mu
    inv = g_ref[...] * lax.rsqrt(var + 1e-5)[None]
    return mu[None], inv


def _k2b_body(z_ref, p_ref, g_ref, be_ref, h_ref, ws_ref, bs_ref, o_ref):
    mu, inv = _bn_from_partials(p_ref[...], g_ref, be_ref)
    bn = (z_ref[...] - mu) * inv + be_ref[...]
    o_ref[...] = (jnp.maximum(bn, 0.0)
                  + jnp.dot(h_ref[...], ws_ref[...],
                            preferred_element_type=jnp.float32)
                  + bs_ref[...])


def _k2b(z, partials, g, be, h_prev, Ws, bs):
    return pl.pallas_call(
        _k2b_body,
        grid=(NB,),
        in_specs=[
            pl.BlockSpec((BR, H), lambda i: (i, 0)),
            pl.BlockSpec((NB, 2, H), lambda i: (0, 0, 0)),
            pl.BlockSpec((1, H), lambda i: (0, 0)),
            pl.BlockSpec((1, H), lambda i: (0, 0)),
            pl.BlockSpec((BR, H), lambda i: (i, 0)),
            pl.BlockSpec((H, H), lambda i: (0, 0)),
            pl.BlockSpec((1, H), lambda i: (0, 0)),
        ],
        out_specs=pl.BlockSpec((BR, H), lambda i: (i, 0)),
        out_shape=jax.ShapeDtypeStruct((N, H), jnp.float32),
    )(z, partials, g, be, h_prev, Ws, bs)


def _kheads_body(z_ref, p_ref, g_ref, be_ref,
                 wp1_ref, bp1_ref, wp2_ref, bp2_ref, wp3_ref, bp3_ref,
                 wd1_ref, bd1_ref, wd2_ref, bd2_ref, wd3_ref, bd3_ref,
                 pp_ref, dd_ref):
    mu, inv = _bn_from_partials(p_ref[...], g_ref, be_ref)
    h4 = jnp.maximum((z_ref[...] - mu) * inv + be_ref[...], 0.0)

    a = jnp.maximum(jnp.dot(h4, wp1_ref[...],
                            preferred_element_type=jnp.float32)
                    + bp1_ref[...], 0.0)
    a = jnp.maximum(jnp.dot(a, wp2_ref[...],
                            preferred_element_type=jnp.float32)
                    + bp2_ref[...], 0.0)
    pp_ref[...] = jax.nn.sigmoid(
        jnp.dot(a, wp3_ref[...], preferred_element_type=jnp.float32)
        + bp3_ref[...])

    d = jnp.maximum(jnp.dot(h4, wd1_ref[...],
                            preferred_element_type=jnp.float32)
                    + bd1_ref[...], 0.0)
    d = jnp.maximum(jnp.dot(d, wd2_ref[...],
                            preferred_element_type=jnp.float32)
                    + bd2_ref[...], 0.0)
    dd_ref[...] = (jnp.dot(d, wd3_ref[...],
                           preferred_element_type=jnp.float32)
                   + bd3_ref[...])


def _kheads(z, partials, g, be, wp1, bp1, wp2, bp2, wp3, bp3,
            wd1, bd1, wd2, bd2, wd3, bd3):
    full = lambda r, c: pl.BlockSpec((r, c), lambda i: (0, 0))
    return pl.pallas_call(
        _kheads_body,
        grid=(NB,),
        in_specs=[
            pl.BlockSpec((BR, H), lambda i: (i, 0)),
            pl.BlockSpec((NB, 2, H), lambda i: (0, 0, 0)),
            full(1, H), full(1, H),
            full(H, H), full(1, H), full(H, H // 2), full(1, H // 2),
            full(H // 2, 1), full(1, 1),
            full(H, H), full(1, H), full(H, H // 2), full(1, H // 2),
            full(H // 2, 1), full(1, 1),
        ],
        out_specs=[
            pl.BlockSpec((BR, 1), lambda i: (i, 0)),
            pl.BlockSpec((BR, 1), lambda i: (i, 0)),
        ],
        out_shape=[
            jax.ShapeDtypeStruct((N, 1), jnp.float32),
            jax.ShapeDtypeStruct((N, 1), jnp.float32),
        ],
    )(z, partials, g, be, wp1, bp1, wp2, bp2, wp3, bp3,
      wd1, bd1, wd2, bd2, wd3, bd3)


def kernel(x, edge_index, edge_weight, pos, W_pos, b_pos, W_in, b_in,
           W_c1, b_c1, W_c2, b_c2, W_c3, b_c3, W_c4, b_c4,
           g1, be1, g2, be2, g3, be3, g4, be4,
           W_s1, b_s1, W_s2, b_s2, W_s3, b_s3,
           Wp1, bp1, Wp2, bp2, Wp3, bp3,
           Wd1, bd1, Wd2, bd2, Wd3, bd3):
    f32 = jnp.float32
    row = lambda v: v.reshape(1, -1)

    padn = EPAD - E
    src = edge_index[0]
    dstp = jnp.concatenate([edge_index[1],
                            jnp.full((padn,), N, jnp.int32)]).reshape(NCHT, CH)
    wp = jnp.concatenate([edge_weight,
                          jnp.zeros((padn,), f32)]).reshape(NCHT, CH)
    srcp = jnp.concatenate([src, jnp.zeros((padn,), jnp.int32)])
    gidx = jnp.concatenate([srcp, srcp + N]).reshape(2 * NCHT, CH)

    posp = jnp.pad(pos, ((0, 0), (0, 6)))
    wposp = jnp.pad(W_pos, ((0, 6), (0, 0)))

    deg = _kdeg(dstp, wp)[:N].reshape(N, 1)
    h = _k0(x, posp, wposp, row(b_pos), W_in, row(b_in))

    layers = [
        (W_c1, b_c1, g1, be1, W_s1, b_s1),
        (W_c2, b_c2, g2, be2, W_s2, b_s2),
        (W_c3, b_c3, g3, be3, W_s3, b_s3),
    ]
    for (Wc, bc, g, be, Ws, bs) in layers:
        hw = _mm1(h, Wc, deg)
        acc = _ksc(hw, gidx, dstp, wp).reshape(2, NPAD, 128)
        z, partials = _k2a(acc, deg, row(bc))
        h = _k2b(z, partials, row(g), row(be), h, Ws, row(bs))

    hw = _mm1(h, W_c4, deg)
    acc = _ksc(hw, gidx, dstp, wp).reshape(2, NPAD, 128)
    z, partials = _k2a(acc, deg, row(b_c4))
    pp, dd = _kheads(z, partials, row(g4), row(be4),
                     Wp1, row(bp1), Wp2, row(bp2), Wp3, row(bp3),
                     Wd1, row(bd1), Wd2, row(bd2), Wd3, row(bd3))
    return (pp[:, 0], dd[:, 0])


# speedup vs baseline: 1.0267x; 1.0267x over previous
import functools

import jax
import jax.numpy as jnp
from jax import lax
from jax.experimental import pallas as pl
from jax.experimental.pallas import tpu as pltpu
from jax.experimental.pallas import tpu_sc as plsc

N = 10000
D = 128
H = 256
E = 320000

NC = 2
NS = 16
L = 16

CH = 128
CPT = 160
GW = 32
NBUF = 2
SPROWS = 10016
EPT = CPT * CH
EPAD = NS * EPT
NCHT = EPAD // CH
NPAD = 11000
DEGP = 11008
ROWS_PT = 640
NB = 10
BR = N // NB

_mesh = plsc.VectorSubcoreMesh(
    core_axis_name="c", subcore_axis_name="s", num_cores=NC, num_subcores=NS)


@functools.partial(
    pl.kernel,
    out_type=jax.ShapeDtypeStruct((DEGP,), jnp.float32),
    mesh=_mesh,
    scratch_types=[
        pltpu.VMEM((CPT, CH), jnp.int32),
        pltpu.VMEM((CPT, CH), jnp.float32),
        pltpu.VMEM((DEGP // NS,), jnp.float32),
        pltpu.VMEM_SHARED((DEGP,), jnp.float32),
    ],
)
def _kdeg(dst_hbm, w_hbm, deg_hbm, dv, wv, zb, spdeg):
    c = lax.axis_index("c")
    s = lax.axis_index("s")

    @pl.when(c == 0)
    def _():
        sl = DEGP // NS
        for i in range(sl // L):
            zb[pl.ds(i * L, L)] = jnp.zeros((L,), jnp.float32)
        pltpu.sync_copy(zb, spdeg.at[pl.ds(s * sl, sl)])
        pltpu.sync_copy(dst_hbm.at[pl.ds(s * CPT, CPT)], dv)
        pltpu.sync_copy(w_hbm.at[pl.ds(s * CPT, CPT)], wv)
        plsc.subcore_barrier()

        def chunk(j, carry):
            pltpu.sync_copy(wv.at[j], spdeg.at[dv.at[j]], add=True)
            return carry

        lax.fori_loop(0, CPT, chunk, 0)
        plsc.subcore_barrier()
        pltpu.sync_copy(spdeg.at[pl.ds(s * sl, sl)], zb)
        pltpu.sync_copy(zb, deg_hbm.at[pl.ds(s * sl, sl)])


@functools.partial(
    pl.kernel,
    out_type=jax.ShapeDtypeStruct((NC * NPAD, 128), jnp.float32),
    mesh=_mesh,
    scratch_types=[
        pltpu.VMEM((GW, CH), jnp.int32),
        pltpu.VMEM((GW, CH), jnp.int32),
        pltpu.VMEM((GW, CH), jnp.float32),
        pltpu.VMEM((NBUF * CH, 128), jnp.float32),
        pltpu.VMEM_SHARED((SPROWS, 128), jnp.float32),
        pltpu.SemaphoreType.DMA,
        pltpu.SemaphoreType.DMA,
    ],
)
def _ksc(hw_hbm, gidx_hbm, dst_hbm, w_hbm, acc_hbm, gv, dv, wv, rbuf,
         spacc, s0, s1):
    sems = (s0, s1)
    c = lax.axis_index("c")
    s = lax.axis_index("s")
    rb = lambda b: rbuf.at[pl.ds(b * CH, CH)]
    rbase = jnp.minimum(s * ROWS_PT, N - ROWS_PT)
    for p in range(ROWS_PT // CH):
        pltpu.sync_copy(hw_hbm.at[pl.ds(c * N + rbase + p * CH, CH)], rb(p % NBUF))
        pltpu.sync_copy(rb(p % NBUF), spacc.at[pl.ds(rbase + p * CH, CH)])
    plsc.subcore_barrier()

    def window(w, carry0):
        cbase = s * CPT + w * GW
        pltpu.sync_copy(gidx_hbm.at[pl.ds(c * NCHT + cbase, GW)], gv)
        pltpu.sync_copy(dst_hbm.at[pl.ds(cbase, GW)], dv)
        pltpu.sync_copy(w_hbm.at[pl.ds(cbase, GW)], wv)
        for b in range(NBUF):
            pltpu.async_copy(hw_hbm.at[gv.at[b]], rb(b), sems[b])

        for p in range(GW):
            b = p % NBUF
            pltpu.make_async_copy(hw_hbm.at[pl.ds(0, CH)], rb(b),
                                  sems[b]).wait()

            def group(q, carry2, p=p, b=b):
                wrow = wv[p, pl.ds(q * L, L)]
                base = q * L
                for t in range(L):
                    wb = jnp.full((L,), wrow[t], jnp.float32)
                    e = base + t
                    for v in range(128 // L):
                        rbuf[b * CH + e, pl.ds(v * L, L)] = (
                            rbuf[b * CH + e, pl.ds(v * L, L)] * wb)
                return carry2

            lax.fori_loop(0, CH // L, group, 0)
            pltpu.sync_copy(rb(b), spacc.at[dv.at[p]], add=True)
            if p + NBUF < GW:
                pltpu.async_copy(hw_hbm.at[gv.at[p + NBUF]], rb(b), sems[b])

        return carry0

    lax.fori_loop(0, CPT // GW, window, 0)
    plsc.subcore_barrier()
    for p in range(ROWS_PT // CH):
        pltpu.sync_copy(spacc.at[pl.ds(rbase + p * CH, CH)], rb(p % NBUF))
        pltpu.sync_copy(rb(p % NBUF),
                        acc_hbm.at[pl.ds(c * NPAD + rbase + p * CH, CH)])


def _k0_body(x_ref, pos_ref, wpos_ref, bpos_ref, win_ref, bin_ref, o_ref):
    pe = jnp.dot(pos_ref[...], wpos_ref[...],
                 preferred_element_type=jnp.float32) + bpos_ref[...]
    h0 = (jnp.dot(x_ref[...], win_ref[0:D],
                  preferred_element_type=jnp.float32)
          + jnp.dot(pe, win_ref[D:D + H],
                    preferred_element_type=jnp.float32)
          + bin_ref[...])
    o_ref[...] = h0


def _k0(x, posp, wposp, bpos, win, bin_):
    return pl.pallas_call(
        _k0_body,
        grid=(NB,),
        in_specs=[
            pl.BlockSpec((BR, D), lambda i: (i, 0)),
            pl.BlockSpec((BR, 8), lambda i: (i, 0)),
            pl.BlockSpec((8, H), lambda i: (0, 0)),
            pl.BlockSpec((1, H), lambda i: (0, 0)),
            pl.BlockSpec((D + H, H), lambda i: (0, 0)),
            pl.BlockSpec((1, H), lambda i: (0, 0)),
        ],
        out_specs=pl.BlockSpec((BR, H), lambda i: (i, 0)),
        out_shape=jax.ShapeDtypeStruct((N, H), jnp.float32),
    )(x, posp, wposp, bpos, win, bin_)


def _mm1_body(h_ref, w_ref, deg_ref, o_ref):
    dis = lax.rsqrt(deg_ref[...] + 1.0)
    o_ref[...] = jnp.dot(h_ref[...], w_ref[...],
                         preferred_element_type=jnp.float32) * dis


def _mm1(h, W, deg):
    return pl.pallas_call(
        _mm1_body,
        grid=(NB, 2),
        in_specs=[
            pl.BlockSpec((BR, H), lambda i, j: (i, 0)),
            pl.BlockSpec((H, 128), lambda i, j: (0, j)),
            pl.BlockSpec((BR, 1), lambda i, j: (i, 0)),
        ],
        out_specs=pl.BlockSpec((BR, 128), lambda i, j: (j * NB + i, 0)),
        out_shape=jax.ShapeDtypeStruct((2 * N, 128), jnp.float32),
    )(h, W, deg)


def _k2a_body(acc_ref, deg_ref, b_ref, z_ref, p_ref):
    dis = lax.rsqrt(deg_ref[...] + 1.0)
    z = jnp.concatenate([acc_ref[0], acc_ref[1]], axis=1) * dis + b_ref[...]
    z_ref[...] = z
    s1 = jnp.sum(z, axis=0)
    s2 = jnp.sum(z * z, axis=0)
    p_ref[...] = jnp.stack([s1, s2])[None]


def _k2a(accv, deg, b):
    return pl.pallas_call(
        _k2a_body,
        grid=(NB,),
        in_specs=[
            pl.BlockSpec((2, BR, 128), lambda i: (0, i, 0)),
            pl.BlockSpec((BR, 1), lambda i: (i, 0)),
            pl.BlockSpec((1, H), lambda i: (0, 0)),
        ],
        out_specs=[
            pl.BlockSpec((BR, H), lambda i: (i, 0)),
            pl.BlockSpec((1, 2, H), lambda i: (i, 0, 0)),
        ],
        out_shape=[
            jax.ShapeDtypeStruct((N, H), jnp.float32),
            jax.ShapeDtypeStruct((NB, 2, H), jnp.float32),
        ],
    )(accv, deg, b)


def _bn_from_partials(p, g_ref, be_ref):
    mu = jnp.sum(p[:, 0, :], axis=0) * (1.0 / N)
    ms = jnp.sum(p[:, 1, :], axis=0) * (1.0 / N)
    var = ms - mu * mu
    inv = g_ref[...] * lax.rsqrt(var + 1e-5)[None]
    return mu[None], inv


def _k2b_body(z_ref, p_ref, g_ref, be_ref, h_ref, ws_ref, bs_ref, o_ref):
    mu, inv = _bn_from_partials(p_ref[...], g_ref, be_ref)
    bn = (z_ref[...] - mu) * inv + be_ref[...]
    o_ref[...] = (jnp.maximum(bn, 0.0)
                  + jnp.dot(h_ref[...], ws_ref[...],
                            preferred_element_type=jnp.float32)
                  + bs_ref[...])


def _k2b(z, partials, g, be, h_prev, Ws, bs):
    return pl.pallas_call(
        _k2b_body,
        grid=(NB,),
        in_specs=[
            pl.BlockSpec((BR, H), lambda i: (i, 0)),
            pl.BlockSpec((NB, 2, H), lambda i: (0, 0, 0)),
            pl.BlockSpec((1, H), lambda i: (0, 0)),
            pl.BlockSpec((1, H), lambda i: (0, 0)),
            pl.BlockSpec((BR, H), lambda i: (i, 0)),
            pl.BlockSpec((H, H), lambda i: (0, 0)),
            pl.BlockSpec((1, H), lambda i: (0, 0)),
        ],
        out_specs=pl.BlockSpec((BR, H), lambda i: (i, 0)),
        out_shape=jax.ShapeDtypeStruct((N, H), jnp.float32),
    )(z, partials, g, be, h_prev, Ws, bs)


def _kheads_body(z_ref, p_ref, g_ref, be_ref,
                 wp1_ref, bp1_ref, wp2_ref, bp2_ref, wp3_ref, bp3_ref,
                 wd1_ref, bd1_ref, wd2_ref, bd2_ref, wd3_ref, bd3_ref,
                 pp_ref, dd_ref):
    mu, inv = _bn_from_partials(p_ref[...], g_ref, be_ref)
    h4 = jnp.maximum((z_ref[...] - mu) * inv + be_ref[...], 0.0)

    a = jnp.maximum(jnp.dot(h4, wp1_ref[...],
                            preferred_element_type=jnp.float32)
                    + bp1_ref[...], 0.0)
    a = jnp.maximum(jnp.dot(a, wp2_ref[...],
                            preferred_element_type=jnp.float32)
                    + bp2_ref[...], 0.0)
    pp_ref[...] = jax.nn.sigmoid(
        jnp.dot(a, wp3_ref[...], preferred_element_type=jnp.float32)
        + bp3_ref[...])

    d = jnp.maximum(jnp.dot(h4, wd1_ref[...],
                            preferred_element_type=jnp.float32)
                    + bd1_ref[...], 0.0)
    d = jnp.maximum(jnp.dot(d, wd2_ref[...],
                            preferred_element_type=jnp.float32)
                    + bd2_ref[...], 0.0)
    dd_ref[...] = (jnp.dot(d, wd3_ref[...],
                           preferred_element_type=jnp.float32)
                   + bd3_ref[...])


def _kheads(z, partials, g, be, wp1, bp1, wp2, bp2, wp3, bp3,
            wd1, bd1, wd2, bd2, wd3, bd3):
    full = lambda r, c: pl.BlockSpec((r, c), lambda i: (0, 0))
    return pl.pallas_call(
        _kheads_body,
        grid=(NB,),
        in_specs=[
            pl.BlockSpec((BR, H), lambda i: (i, 0)),
            pl.BlockSpec((NB, 2, H), lambda i: (0, 0, 0)),
            full(1, H), full(1, H),
            full(H, H), full(1, H), full(H, H // 2), full(1, H // 2),
            full(H // 2, 1), full(1, 1),
            full(H, H), full(1, H), full(H, H // 2), full(1, H // 2),
            full(H // 2, 1), full(1, 1),
        ],
        out_specs=[
            pl.BlockSpec((BR, 1), lambda i: (i, 0)),
            pl.BlockSpec((BR, 1), lambda i: (i, 0)),
        ],
        out_shape=[
            jax.ShapeDtypeStruct((N, 1), jnp.float32),
            jax.ShapeDtypeStruct((N, 1), jnp.float32),
        ],
    )(z, partials, g, be, wp1, bp1, wp2, bp2, wp3, bp3,
      wd1, bd1, wd2, bd2, wd3, bd3)


def kernel(x, edge_index, edge_weight, pos, W_pos, b_pos, W_in, b_in,
           W_c1, b_c1, W_c2, b_c2, W_c3, b_c3, W_c4, b_c4,
           g1, be1, g2, be2, g3, be3, g4, be4,
           W_s1, b_s1, W_s2, b_s2, W_s3, b_s3,
           Wp1, bp1, Wp2, bp2, Wp3, bp3,
           Wd1, bd1, Wd2, bd2, Wd3, bd3):
    f32 = jnp.float32
    row = lambda v: v.reshape(1, -1)

    padn = EPAD - E
    src = edge_index[0]
    dstp = jnp.concatenate([edge_index[1],
                            jnp.full((padn,), N, jnp.int32)]).reshape(NCHT, CH)
    wp = jnp.concatenate([edge_weight,
                          jnp.zeros((padn,), f32)]).reshape(NCHT, CH)
    srcp = jnp.concatenate([src, jnp.zeros((padn,), jnp.int32)])
    gidx = jnp.concatenate([srcp, srcp + N]).reshape(2 * NCHT, CH)

    posp = jnp.pad(pos, ((0, 0), (0, 6)))
    wposp = jnp.pad(W_pos, ((0, 6), (0, 0)))

    deg = _kdeg(dstp, wp)[:N].reshape(N, 1)
    h = _k0(x, posp, wposp, row(b_pos), W_in, row(b_in))

    layers = [
        (W_c1, b_c1, g1, be1, W_s1, b_s1),
        (W_c2, b_c2, g2, be2, W_s2, b_s2),
        (W_c3, b_c3, g3, be3, W_s3, b_s3),
    ]
    for (Wc, bc, g, be, Ws, bs) in layers:
        hw = _mm1(h, Wc, deg)
        acc = _ksc(hw, gidx, dstp, wp).reshape(2, NPAD, 128)
        z, partials = _k2a(acc, deg, row(bc))
        h = _k2b(z, partials, row(g), row(be), h, Ws, row(bs))

    hw = _mm1(h, W_c4, deg)
    acc = _ksc(hw, gidx, dstp, wp).reshape(2, NPAD, 128)
    z, partials = _k2a(acc, deg, row(b_c4))
    pp, dd = _kheads(z, partials, row(g4), row(be4),
                     Wp1, row(bp1), Wp2, row(bp2), Wp3, row(bp3),
                     Wd1, row(bd1), Wd2, row(bd2), Wd3, row(bd3))
    return (pp[:, 0], dd[:, 0])


# speedup vs baseline: 1.1354x; 1.1059x over previous
import functools

import jax
import jax.numpy as jnp
from jax import lax
from jax.experimental import pallas as pl
from jax.experimental.pallas import tpu as pltpu
from jax.experimental.pallas import tpu_sc as plsc

N = 10000
D = 128
H = 256
E = 320000

NC = 2
NS = 16
L = 16

CH = 64
CPT = 320
GW = 32
NBUF = 4
SPROWS = 10016
EPT = CPT * CH
EPAD = NS * EPT
NCHT = EPAD // CH
NPAD = 11000
DEGP = 11008
ROWS_PT = 640
NB = 10
BR = N // NB

_mesh = plsc.VectorSubcoreMesh(
    core_axis_name="c", subcore_axis_name="s", num_cores=NC, num_subcores=NS)


@functools.partial(
    pl.kernel,
    out_type=jax.ShapeDtypeStruct((DEGP,), jnp.float32),
    mesh=_mesh,
    scratch_types=[
        pltpu.VMEM((CPT, CH), jnp.int32),
        pltpu.VMEM((CPT, CH), jnp.float32),
        pltpu.VMEM((DEGP // NS,), jnp.float32),
        pltpu.VMEM_SHARED((DEGP,), jnp.float32),
    ],
)
def _kdeg(dst_hbm, w_hbm, deg_hbm, dv, wv, zb, spdeg):
    c = lax.axis_index("c")
    s = lax.axis_index("s")

    @pl.when(c == 0)
    def _():
        sl = DEGP // NS
        for i in range(sl // L):
            zb[pl.ds(i * L, L)] = jnp.zeros((L,), jnp.float32)
        pltpu.sync_copy(zb, spdeg.at[pl.ds(s * sl, sl)])
        pltpu.sync_copy(dst_hbm.at[pl.ds(s * CPT, CPT)], dv)
        pltpu.sync_copy(w_hbm.at[pl.ds(s * CPT, CPT)], wv)
        plsc.subcore_barrier()

        def chunk(j, carry):
            pltpu.sync_copy(wv.at[j], spdeg.at[dv.at[j]], add=True)
            return carry

        lax.fori_loop(0, CPT, chunk, 0)
        plsc.subcore_barrier()
        pltpu.sync_copy(spdeg.at[pl.ds(s * sl, sl)], zb)
        pltpu.sync_copy(zb, deg_hbm.at[pl.ds(s * sl, sl)])


@functools.partial(
    pl.kernel,
    out_type=jax.ShapeDtypeStruct((NC * NPAD, 128), jnp.float32),
    mesh=_mesh,
    scratch_types=[
        pltpu.VMEM((GW, CH), jnp.int32),
        pltpu.VMEM((GW, CH), jnp.int32),
        pltpu.VMEM((GW, CH), jnp.float32),
        pltpu.VMEM((NBUF * CH, 128), jnp.float32),
        pltpu.VMEM_SHARED((SPROWS, 128), jnp.float32),
        pltpu.SemaphoreType.DMA,
        pltpu.SemaphoreType.DMA,
        pltpu.SemaphoreType.DMA,
        pltpu.SemaphoreType.DMA,
        pltpu.SemaphoreType.DMA,
        pltpu.SemaphoreType.DMA,
        pltpu.SemaphoreType.DMA,
        pltpu.SemaphoreType.DMA,
    ],
)
def _ksc(hw_hbm, gidx_hbm, dst_hbm, w_hbm, acc_hbm, gv, dv, wv, rbuf,
         spacc, g0, g1, g2, g3, t0, t1, t2, t3):
    gsem = (g0, g1, g2, g3)
    ssem = (t0, t1, t2, t3)
    c = lax.axis_index("c")
    s = lax.axis_index("s")
    rb = lambda b: rbuf.at[pl.ds(b * CH, CH)]
    rbase = jnp.minimum(s * ROWS_PT, N - ROWS_PT)
    for p in range(ROWS_PT // CH):
        pltpu.sync_copy(hw_hbm.at[pl.ds(c * N + rbase + p * CH, CH)], rb(p % NBUF))
        pltpu.sync_copy(rb(p % NBUF), spacc.at[pl.ds(rbase + p * CH, CH)])
    plsc.subcore_barrier()

    def window(w, carry0):
        cbase = s * CPT + w * GW
        pltpu.sync_copy(gidx_hbm.at[pl.ds(c * NCHT + cbase, GW)], gv)
        pltpu.sync_copy(dst_hbm.at[pl.ds(cbase, GW)], dv)
        pltpu.sync_copy(w_hbm.at[pl.ds(cbase, GW)], wv)

        def swait(b):
            pltpu.make_async_copy(hw_hbm.at[pl.ds(0, CH)], rb(b),
                                  ssem[b]).wait()

        for b in range(2):
            @pl.when(w > 0)
            def _(b=b):
                swait(b)
            pltpu.async_copy(hw_hbm.at[gv.at[b]], rb(b), gsem[b])

        for p in range(GW):
            if p + 2 < GW:
                bq = (p + 2) % NBUF

                def start(bq=bq, p=p):
                    swait(bq)
                    pltpu.async_copy(hw_hbm.at[gv.at[p + 2]], rb(bq),
                                     gsem[bq])

                if p < 2:
                    @pl.when(w > 0)
                    def _(start=start):
                        start()

                    @pl.when(w == 0)
                    def _(bq=bq, p=p):
                        pltpu.async_copy(hw_hbm.at[gv.at[p + 2]], rb(bq),
                                         gsem[bq])
                else:
                    start()

            b = p % NBUF
            pltpu.make_async_copy(hw_hbm.at[pl.ds(0, CH)], rb(b),
                                  gsem[b]).wait()

            def group(q, carry2, p=p, b=b):
                wrow = wv[p, pl.ds(q * L, L)]
                base = q * L
                for t in range(L):
                    wb = jnp.full((L,), wrow[t], jnp.float32)
                    e = base + t
                    for v in range(128 // L):
                        rbuf[b * CH + e, pl.ds(v * L, L)] = (
                            rbuf[b * CH + e, pl.ds(v * L, L)] * wb)
                return carry2

            lax.fori_loop(0, CH // L, group, 0)
            pltpu.async_copy(rb(b), spacc.at[dv.at[p]], ssem[b], add=True)

        return carry0

    lax.fori_loop(0, CPT // GW, window, 0)
    for b in range(NBUF):
        pltpu.make_async_copy(hw_hbm.at[pl.ds(0, CH)], rb(b), ssem[b]).wait()
    plsc.subcore_barrier()
    for p in range(ROWS_PT // CH):
        pltpu.sync_copy(spacc.at[pl.ds(rbase + p * CH, CH)], rb(p % NBUF))
        pltpu.sync_copy(rb(p % NBUF),
                        acc_hbm.at[pl.ds(c * NPAD + rbase + p * CH, CH)])


def _k0_body(x_ref, pos_ref, wpos_ref, bpos_ref, win_ref, bin_ref, o_ref):
    pe = jnp.dot(pos_ref[...], wpos_ref[...],
                 preferred_element_type=jnp.float32) + bpos_ref[...]
    h0 = (jnp.dot(x_ref[...], win_ref[0:D],
                  preferred_element_type=jnp.float32)
          + jnp.dot(pe, win_ref[D:D + H],
                    preferred_element_type=jnp.float32)
          + bin_ref[...])
    o_ref[...] = h0


def _k0(x, posp, wposp, bpos, win, bin_):
    return pl.pallas_call(
        _k0_body,
        grid=(NB,),
        in_specs=[
            pl.BlockSpec((BR, D), lambda i: (i, 0)),
            pl.BlockSpec((BR, 8), lambda i: (i, 0)),
            pl.BlockSpec((8, H), lambda i: (0, 0)),
            pl.BlockSpec((1, H), lambda i: (0, 0)),
            pl.BlockSpec((D + H, H), lambda i: (0, 0)),
            pl.BlockSpec((1, H), lambda i: (0, 0)),
        ],
        out_specs=pl.BlockSpec((BR, H), lambda i: (i, 0)),
        out_shape=jax.ShapeDtypeStruct((N, H), jnp.float32),
    )(x, posp, wposp, bpos, win, bin_)


def _mm1_body(h_ref, w_ref, deg_ref, o_ref):
    dis = lax.rsqrt(deg_ref[...] + 1.0)
    o_ref[...] = jnp.dot(h_ref[...], w_ref[...],
                         preferred_element_type=jnp.float32) * dis


def _mm1(h, W, deg):
    return pl.pallas_call(
        _mm1_body,
        grid=(NB, 2),
        in_specs=[
            pl.BlockSpec((BR, H), lambda i, j: (i, 0)),
            pl.BlockSpec((H, 128), lambda i, j: (0, j)),
            pl.BlockSpec((BR, 1), lambda i, j: (i, 0)),
        ],
        out_specs=pl.BlockSpec((BR, 128), lambda i, j: (j * NB + i, 0)),
        out_shape=jax.ShapeDtypeStruct((2 * N, 128), jnp.float32),
    )(h, W, deg)


def _k2a_body(acc_ref, deg_ref, b_ref, z_ref, p_ref):
    dis = lax.rsqrt(deg_ref[...] + 1.0)
    z = jnp.concatenate([acc_ref[0], acc_ref[1]], axis=1) * dis + b_ref[...]
    z_ref[...] = z
    s1 = jnp.sum(z, axis=0)
    s2 = jnp.sum(z * z, axis=0)
    p_ref[...] = jnp.stack([s1, s2])[None]


def _k2a(accv, deg, b):
    return pl.pallas_call(
        _k2a_body,
        grid=(NB,),
        in_specs=[
            pl.BlockSpec((2, BR, 128), lambda i: (0, i, 0)),
            pl.BlockSpec((BR, 1), lambda i: (i, 0)),
            pl.BlockSpec((1, H), lambda i: (0, 0)),
        ],
        out_specs=[
            pl.BlockSpec((BR, H), lambda i: (i, 0)),
            pl.BlockSpec((1, 2, H), lambda i: (i, 0, 0)),
        ],
        out_shape=[
            jax.ShapeDtypeStruct((N, H), jnp.float32),
            jax.ShapeDtypeStruct((NB, 2, H), jnp.float32),
        ],
    )(accv, deg, b)


def _bn_from_partials(p, g_ref, be_ref):
    mu = jnp.sum(p[:, 0, :], axis=0) * (1.0 / N)
    ms = jnp.sum(p[:, 1, :], axis=0) * (1.0 / N)
    var = ms - mu * mu
    inv = g_ref[...] * lax.rsqrt(var + 1e-5)[None]
    return mu[None], inv


def _k2b_body(z_ref, p_ref, g_ref, be_ref, h_ref, ws_ref, bs_ref, o_ref):
    mu, inv = _bn_from_partials(p_ref[...], g_ref, be_ref)
    bn = (z_ref[...] - mu) * inv + be_ref[...]
    o_ref[...] = (jnp.maximum(bn, 0.0)
                  + jnp.dot(h_ref[...], ws_ref[...],
                            preferred_element_type=jnp.float32)
                  + bs_ref[...])


def _k2b(z, partials, g, be, h_prev, Ws, bs):
    return pl.pallas_call(
        _k2b_body,
        grid=(NB,),
        in_specs=[
            pl.BlockSpec((BR, H), lambda i: (i, 0)),
            pl.BlockSpec((NB, 2, H), lambda i: (0, 0, 0)),
            pl.BlockSpec((1, H), lambda i: (0, 0)),
            pl.BlockSpec((1, H), lambda i: (0, 0)),
            pl.BlockSpec((BR, H), lambda i: (i, 0)),
            pl.BlockSpec((H, H), lambda i: (0, 0)),
            pl.BlockSpec((1, H), lambda i: (0, 0)),
        ],
        out_specs=pl.BlockSpec((BR, H), lambda i: (i, 0)),
        out_shape=jax.ShapeDtypeStruct((N, H), jnp.float32),
    )(z, partials, g, be, h_prev, Ws, bs)


def _kheads_body(z_ref, p_ref, g_ref, be_ref,
                 wp1_ref, bp1_ref, wp2_ref, bp2_ref, wp3_ref, bp3_ref,
                 wd1_ref, bd1_ref, wd2_ref, bd2_ref, wd3_ref, bd3_ref,
                 pp_ref, dd_ref):
    mu, inv = _bn_from_partials(p_ref[...], g_ref, be_ref)
    h4 = jnp.maximum((z_ref[...] - mu) * inv + be_ref[...], 0.0)

    a = jnp.maximum(jnp.dot(h4, wp1_ref[...],
                            preferred_element_type=jnp.float32)
                    + bp1_ref[...], 0.0)
    a = jnp.maximum(jnp.dot(a, wp2_ref[...],
                            preferred_element_type=jnp.float32)
                    + bp2_ref[...], 0.0)
    pp_ref[...] = jax.nn.sigmoid(
        jnp.dot(a, wp3_ref[...], preferred_element_type=jnp.float32)
        + bp3_ref[...])

    d = jnp.maximum(jnp.dot(h4, wd1_ref[...],
                            preferred_element_type=jnp.float32)
                    + bd1_ref[...], 0.0)
    d = jnp.maximum(jnp.dot(d, wd2_ref[...],
                            preferred_element_type=jnp.float32)
                    + bd2_ref[...], 0.0)
    dd_ref[...] = (jnp.dot(d, wd3_ref[...],
                           preferred_element_type=jnp.float32)
                   + bd3_ref[...])


def _kheads(z, partials, g, be, wp1, bp1, wp2, bp2, wp3, bp3,
            wd1, bd1, wd2, bd2, wd3, bd3):
    full = lambda r, c: pl.BlockSpec((r, c), lambda i: (0, 0))
    return pl.pallas_call(
        _kheads_body,
        grid=(NB,),
        in_specs=[
            pl.BlockSpec((BR, H), lambda i: (i, 0)),
            pl.BlockSpec((NB, 2, H), lambda i: (0, 0, 0)),
            full(1, H), full(1, H),
            full(H, H), full(1, H), full(H, H // 2), full(1, H // 2),
            full(H // 2, 1), full(1, 1),
            full(H, H), full(1, H), full(H, H // 2), full(1, H // 2),
            full(H // 2, 1), full(1, 1),
        ],
        out_specs=[
            pl.BlockSpec((BR, 1), lambda i: (i, 0)),
            pl.BlockSpec((BR, 1), lambda i: (i, 0)),
        ],
        out_shape=[
            jax.ShapeDtypeStruct((N, 1), jnp.float32),
            jax.ShapeDtypeStruct((N, 1), jnp.float32),
        ],
    )(z, partials, g, be, wp1, bp1, wp2, bp2, wp3, bp3,
      wd1, bd1, wd2, bd2, wd3, bd3)


def kernel(x, edge_index, edge_weight, pos, W_pos, b_pos, W_in, b_in,
           W_c1, b_c1, W_c2, b_c2, W_c3, b_c3, W_c4, b_c4,
           g1, be1, g2, be2, g3, be3, g4, be4,
           W_s1, b_s1, W_s2, b_s2, W_s3, b_s3,
           Wp1, bp1, Wp2, bp2, Wp3, bp3,
           Wd1, bd1, Wd2, bd2, Wd3, bd3):
    f32 = jnp.float32
    row = lambda v: v.reshape(1, -1)

    padn = EPAD - E
    src = edge_index[0]
    dstp = jnp.concatenate([edge_index[1],
                            jnp.full((padn,), N, jnp.int32)]).reshape(NCHT, CH)
    wp = jnp.concatenate([edge_weight,
                          jnp.zeros((padn,), f32)]).reshape(NCHT, CH)
    srcp = jnp.concatenate([src, jnp.zeros((padn,), jnp.int32)])
    gidx = jnp.concatenate([srcp, srcp + N]).reshape(2 * NCHT, CH)

    posp = jnp.pad(pos, ((0, 0), (0, 6)))
    wposp = jnp.pad(W_pos, ((0, 6), (0, 0)))

    deg = _kdeg(dstp, wp)[:N].reshape(N, 1)
    h = _k0(x, posp, wposp, row(b_pos), W_in, row(b_in))

    layers = [
        (W_c1, b_c1, g1, be1, W_s1, b_s1),
        (W_c2, b_c2, g2, be2, W_s2, b_s2),
        (W_c3, b_c3, g3, be3, W_s3, b_s3),
    ]
    for (Wc, bc, g, be, Ws, bs) in layers:
        hw = _mm1(h, Wc, deg)
        acc = _ksc(hw, gidx, dstp, wp).reshape(2, NPAD, 128)
        z, partials = _k2a(acc, deg, row(bc))
        h = _k2b(z, partials, row(g), row(be), h, Ws, row(bs))

    hw = _mm1(h, W_c4, deg)
    acc = _ksc(hw, gidx, dstp, wp).reshape(2, NPAD, 128)
    z, partials = _k2a(acc, deg, row(b_c4))
    pp, dd = _kheads(z, partials, row(g4), row(be4),
                     Wp1, row(bp1), Wp2, row(bp2), Wp3, row(bp3),
                     Wd1, row(bd1), Wd2, row(bd2), Wd3, row(bd3))
    return (pp[:, 0], dd[:, 0])
